# baseline (speedup 1.0000x reference)
import functools

import jax
import jax.numpy as jnp
from jax import lax
from jax.experimental import pallas as pl
from jax.experimental.pallas import tpu as pltpu
from jax.experimental.pallas import tpu_sc as plsc

NU = 10000
NI = 10000
D = 128
NE = 320000
NC = 2
NS = 16
NW = NC * NS
CHUNK = 128
NCHUNKS = NE // CHUNK
ROWS_PER_TILE = NU // NS
EDGE_BIAS = 0.5
N_LAYERS = 2

_mesh = plsc.VectorSubcoreMesh(
    core_axis_name="c", subcore_axis_name="s", num_cores=NC, num_subcores=NS)


def _wid_and_chunks():
  sid = lax.axis_index("s")
  cid = lax.axis_index("c")
  wid = sid * NC + cid
  nfull = NCHUNKS // NW
  rem = NCHUNKS - nfull * NW
  nchunks = nfull + jnp.where(wid < rem, 1, 0)
  return sid, cid, wid, nchunks


_BCH = 26
_NBLK = 3
_BE = _BCH * CHUNK


@functools.partial(
    pl.kernel,
    out_type=[
        jax.ShapeDtypeStruct((NE,), jnp.float32),
    ],
    mesh=_mesh,
    compiler_params=pltpu.CompilerParams(needs_layout_passes=False),
    scratch_types=[
        pltpu.VMEM((_BE,), jnp.int32),
        pltpu.VMEM((_BE,), jnp.int32),
        pltpu.VMEM((_BE,), jnp.float32),
        pltpu.VMEM((CHUNK, D), jnp.float32),
        pltpu.VMEM((CHUNK, D), jnp.float32),
        pltpu.VMEM((CHUNK, D), jnp.float32),
        pltpu.VMEM((CHUNK, D), jnp.float32),
        pltpu.VMEM((D,), jnp.float32),
        pltpu.VMEM((256,), jnp.float32),
        pltpu.VMEM((16,), jnp.float32),
        pltpu.SemaphoreType.DMA,
        pltpu.SemaphoreType.DMA,
    ],
)
def _sc_gate(a_hbm, b_hbm, w2_hbm, consts_hbm, src_hbm, dst_hbm,
             g_hbm,
             srcb, dstb, gblk, bufA0, bufB0, bufA1, bufB1,
             w2_buf, dot_buf, cbuf, sem0, sem1):
  sid, cid, wid, _ = _wid_and_chunks()
  cstart = wid * 78 + jnp.minimum(wid, 4)
  has_tail = wid < 4

  pltpu.sync_copy(w2_hbm, w2_buf)
  pltpu.sync_copy(consts_hbm, cbuf)

  b2vec = cbuf[...]
  iota16 = lax.iota(jnp.int32, 16)
  w2vecs = [w2_buf[pl.ds(kb * 16, 16)] for kb in range(8)]

  def issue(jrel, bA, bB, sem):
    off = pl.ds(jrel * CHUNK, CHUNK)
    pltpu.async_copy(a_hbm.at[srcb.at[off]], bA, sem)
    pltpu.async_copy(b_hbm.at[dstb.at[off]], bB, sem)

  def wait(bA, bB, sem):
    pltpu.make_async_copy(a_hbm.at[pl.ds(0, CHUNK)], bA, sem).wait()
    pltpu.make_async_copy(b_hbm.at[pl.ds(0, CHUNK)], bB, sem).wait()

  def compute(bA, bB, jrel):
    goff = jrel * CHUNK
    for gr in range(8):

      def e_body(t4, carry2):
        for q in range(4):
          t = t4 * 4 + q
          e = gr * 16 + t
          acc0 = jnp.zeros((16,), jnp.float32)
          acc1 = jnp.zeros((16,), jnp.float32)
          for kb in range(0, 8, 2):
            sl = pl.ds(kb * 16, 16)
            h = jnp.maximum(bA[e, sl] + bB[e, sl], 0.0)
            acc0 = acc0 + h * w2vecs[kb]
            sl = pl.ds((kb + 1) * 16, 16)
            h = jnp.maximum(bA[e, sl] + bB[e, sl], 0.0)
            acc1 = acc1 + h * w2vecs[kb + 1]
          dot_buf[pl.ds(t * 16, 16)] = acc0 + acc1
        return carry2

      lax.fori_loop(0, 4, e_body, 0)
      tot = jnp.zeros((16,), jnp.float32)
      for t2 in range(16):
        tot = tot + plsc.load_gather(dot_buf, [iota16 * 16 + t2])
      logit5 = (tot + b2vec) * 5.0
      g = 1.0 / (1.0 + jnp.exp(-logit5)) + EDGE_BIAS
      gblk[pl.ds(goff + gr * 16, 16)] = g

  def block_body(b, carry):
    bc = cstart + b * _BCH
    pltpu.sync_copy(src_hbm.at[pl.ds(bc * CHUNK, _BE)], srcb)
    pltpu.sync_copy(dst_hbm.at[pl.ds(bc * CHUNK, _BE)], dstb)
    issue(0, bufA0, bufB0, sem0)

    def pair_body(p, c2):
      j0 = p * 2
      j1 = j0 + 1
      issue(j1, bufA1, bufB1, sem1)
      wait(bufA0, bufB0, sem0)
      compute(bufA0, bufB0, j0)

      @pl.when(j1 + 1 < _BCH)
      def _nxt():
        issue(j1 + 1, bufA0, bufB0, sem0)

      wait(bufA1, bufB1, sem1)
      compute(bufA1, bufB1, j1)
      return c2

    lax.fori_loop(0, _BCH // 2, pair_body, 0)
    pltpu.sync_copy(gblk, g_hbm.at[pl.ds(bc * CHUNK, _BE)])
    return carry

  lax.fori_loop(0, _NBLK, block_body, 0)

  @pl.when(has_tail)
  def _tail():
    tb = (cstart + _NBLK * _BCH) * CHUNK
    pltpu.sync_copy(src_hbm.at[pl.ds(tb, CHUNK)], srcb.at[pl.ds(0, CHUNK)])
    pltpu.sync_copy(dst_hbm.at[pl.ds(tb, CHUNK)], dstb.at[pl.ds(0, CHUNK)])
    issue(0, bufA0, bufB0, sem0)
    wait(bufA0, bufB0, sem0)
    compute(bufA0, bufB0, 0)
    pltpu.sync_copy(gblk.at[pl.ds(0, CHUNK)], g_hbm.at[pl.ds(tb, CHUNK)])


def _make_segsum(scaled: bool, extra: str | None = None):
  with_extra = extra is not None
  BCH = 26
  NBLK = 3
  BE = BCH * CHUNK

  scratch = [
      pltpu.VMEM((BE,), jnp.int32),
      pltpu.VMEM((CHUNK,), jnp.int32),
      pltpu.VMEM((CHUNK,), jnp.int32),
      pltpu.VMEM((CHUNK, D), jnp.float32),
      pltpu.VMEM((CHUNK, D), jnp.float32),
      pltpu.VMEM((BE,), jnp.float32),
      pltpu.VMEM((CHUNK,), jnp.float32),
      pltpu.VMEM_SHARED((NU, D), jnp.float32),
  ] + ([pltpu.VMEM_SHARED((NU,), jnp.float32)] if with_extra else []) + [
      pltpu.SemaphoreType.DMA,
      pltpu.SemaphoreType.DMA,
  ]

  NUNITS = NU // CHUNK

  def body(*refs):
    (table_hbm, gidx_hbm, seg_hbm, g_all_hbm, zrows_hbm) = refs[:5]
    refs = refs[5:]
    if with_extra:
      zeros1d_hbm, part_out, extra_out = refs[:3]
      refs = refs[3:]
    else:
      part_out = refs[0]
      refs = refs[1:]
    gidx, segb0, segb1, rows0, rows1, gblk, ones_buf, acc = refs[:8]
    refs = refs[8:]
    if with_extra:
      extra_acc = refs[0]
      refs = refs[1:]
    sem0, sem1 = refs
    sid, cid, wid, _ = _wid_and_chunks()
    nunits = NUNITS // NS + jnp.where(sid < NUNITS % NS, 1, 0)
    cstart = wid * 78 + jnp.minimum(wid, 4)
    has_tail = wid < 4

    pltpu.sync_copy(zrows_hbm, rows0)

    def zero_body(k, c2):
      u0 = (sid + k * NS) * CHUNK
      pltpu.sync_copy(rows0, acc.at[pl.ds(u0, CHUNK)])
      return c2

    lax.fori_loop(0, nunits, zero_body, 0)

    @pl.when(sid == 15)
    def _ztail():
      pltpu.sync_copy(rows0.at[pl.ds(0, 16)], acc.at[pl.ds(NUNITS * CHUNK, 16)])

    if with_extra:
      if extra == 'deg':
        for kb in range(8):
          ones_buf[pl.ds(kb * 16, 16)] = jnp.ones((16,), jnp.float32)

      @pl.when(sid == 0)
      def _zextra():
        pltpu.sync_copy(zeros1d_hbm, extra_acc)

    plsc.subcore_barrier()

    def extra_scatter(segb, jrel):
      if extra == 'deg':
        pltpu.sync_copy(ones_buf, extra_acc.at[segb], add=True)
      elif extra == 'rs':
        pltpu.sync_copy(gblk.at[pl.ds(jrel * CHUNK, CHUNK)],
                        extra_acc.at[segb], add=True)

    def issue(bc, jrel, rbuf, sem):
      pltpu.async_copy(table_hbm.at[gidx.at[pl.ds(jrel * CHUNK, CHUNK)]],
                       rbuf, sem)

    def wait(rbuf, sem):
      pltpu.make_async_copy(table_hbm.at[pl.ds(0, CHUNK)], rbuf, sem).wait()

    def scale(rbuf, jrel):
      if not scaled:
        return
      goff = jrel * CHUNK

      def scale_body(p, c2):
        for q in range(4):
          e = p * 4 + q
          ge = plsc.load_gather(gblk, [jnp.full((16,), 0, jnp.int32) + goff + e])
          for kb in range(8):
            sl = pl.ds(kb * 16, 16)
            rbuf[e, sl] = rbuf[e, sl] * ge
        return c2

      lax.fori_loop(0, CHUNK // 4, scale_body, 0)

    def block_body(b, carry):
      bc = cstart + b * BCH
      pltpu.sync_copy(gidx_hbm.at[pl.ds(bc * CHUNK, BE)], gidx)
      if scaled:
        pltpu.sync_copy(g_all_hbm.at[pl.ds(bc * CHUNK, BE)], gblk)
      issue(bc, 0, rows0, sem0)

      def pair_body(p, c2):
        j0 = p * 2
        j1 = j0 + 1
        issue(bc, j1, rows1, sem1)
        pltpu.sync_copy(seg_hbm.at[pl.ds((bc + j0) * CHUNK, CHUNK)], segb0)
        wait(rows0, sem0)
        scale(rows0, j0)
        pltpu.sync_copy(rows0, acc.at[segb0], add=True)
        if with_extra:
          extra_scatter(segb0, j0)

        @pl.when(j1 + 1 < BCH)
        def _nxt():
          issue(bc, j1 + 1, rows0, sem0)

        pltpu.sync_copy(seg_hbm.at[pl.ds((bc + j1) * CHUNK, CHUNK)], segb1)
        wait(rows1, sem1)
        scale(rows1, j1)
        pltpu.sync_copy(rows1, acc.at[segb1], add=True)
        if with_extra:
          extra_scatter(segb1, j1)
        return c2

      lax.fori_loop(0, BCH // 2, pair_body, 0)
      return carry

    lax.fori_loop(0, NBLK, block_body, 0)

    @pl.when(has_tail)
    def _tail():
      tb = (cstart + NBLK * BCH) * CHUNK
      pltpu.sync_copy(gidx_hbm.at[pl.ds(tb, CHUNK)], gidx.at[pl.ds(0, CHUNK)])
      if scaled:
        pltpu.sync_copy(g_all_hbm.at[pl.ds(tb, CHUNK)], gblk.at[pl.ds(0, CHUNK)])
      issue(0, 0, rows0, sem0)
      pltpu.sync_copy(seg_hbm.at[pl.ds(tb, CHUNK)], segb0)
      wait(rows0, sem0)
      scale(rows0, 0)
      pltpu.sync_copy(rows0, acc.at[segb0], add=True)
      if with_extra:
        extra_scatter(segb0, 0)

    plsc.subcore_barrier()

    def drain_body(k, c2):
      u0 = (sid + k * NS) * CHUNK
      pltpu.sync_copy(acc.at[pl.ds(u0, CHUNK)], rows0)
      pltpu.sync_copy(rows0, part_out.at[cid, pl.ds(u0, CHUNK)])
      return c2

    lax.fori_loop(0, nunits, drain_body, 0)

    @pl.when(sid == 15)
    def _dtail():
      pltpu.sync_copy(acc.at[pl.ds(NUNITS * CHUNK, 16)], rows0.at[pl.ds(0, 16)])
      pltpu.sync_copy(rows0.at[pl.ds(0, 16)],
                      part_out.at[cid, pl.ds(NUNITS * CHUNK, 16)])

    if with_extra:
      @pl.when(sid == 0)
      def _dextra():
        pltpu.sync_copy(extra_acc, extra_out.at[cid])

  out_type = [jax.ShapeDtypeStruct((NC, NU, D), jnp.float32)]
  if with_extra:
    out_type.append(jax.ShapeDtypeStruct((NC, NU), jnp.float32))
  return functools.partial(
      pl.kernel,
      out_type=out_type,
      mesh=_mesh,
      compiler_params=pltpu.CompilerParams(needs_layout_passes=False),
      scratch_types=scratch,
  )(body)


_sc_segsum_scaled = _make_segsum(True)
_sc_segsum_scaled_rs = _make_segsum(True, extra='rs')
_sc_segsum_plain = _make_segsum(False)
_sc_segsum_plain_deg = _make_segsum(False, extra='deg')


_RB = 1000


def _ab_body(x_ref, wa_ref, wb_ref, b1_ref, a_ref, b_ref):
  x = x_ref[...]
  a_ref[...] = jnp.dot(x, wa_ref[...],
                       preferred_element_type=jnp.float32) + b1_ref[...][None, :]
  b_ref[...] = jnp.dot(x, wb_ref[...], preferred_element_type=jnp.float32)


def _ab_matmul(x, wa, wb, b1):
  return pl.pallas_call(
      _ab_body,
      grid=(NU // _RB,),
      in_specs=[
          pl.BlockSpec((_RB, D), lambda i: (i, 0)),
          pl.BlockSpec((D, D), lambda i: (0, 0)),
          pl.BlockSpec((D, D), lambda i: (0, 0)),
          pl.BlockSpec((D,), lambda i: (0,)),
      ],
      out_specs=[
          pl.BlockSpec((_RB, D), lambda i: (i, 0)),
          pl.BlockSpec((_RB, D), lambda i: (i, 0)),
      ],
      out_shape=[
          jax.ShapeDtypeStruct((NU, D), jnp.float32),
          jax.ShapeDtypeStruct((NU, D), jnp.float32),
      ],
  )(x, wa, wb, b1)


def _prep_body(rs_ref, ud_ref, id_ref, inv_ref, uinv_ref, iinv_ref):
  def inv_of(ref):
    t = ref[0, :] + ref[1, :]
    return jnp.where(t > 0, 1.0 / t, 0.0)

  inv_ref[...] = inv_of(rs_ref)
  uinv_ref[...] = inv_of(ud_ref)
  iinv_ref[...] = inv_of(id_ref)


def _prep(rs_p, ud_p, id_p):
  return pl.pallas_call(
      _prep_body,
      out_shape=[
          jax.ShapeDtypeStruct((NU,), jnp.float32),
          jax.ShapeDtypeStruct((NU,), jnp.float32),
          jax.ShapeDtypeStruct((NI,), jnp.float32),
      ],
  )(rs_p, ud_p, id_p)


def _combine_body(ps_ref, pr_ref, pi_ref, inv_ref, uinv_ref, iinv_ref,
                  u_ref, i_ref):
  u_ref[...] = (inv_ref[...] * (ps_ref[0] + ps_ref[1])
                + uinv_ref[...] * (pr_ref[0] + pr_ref[1]))
  i_ref[...] = iinv_ref[...] * (pi_ref[0] + pi_ref[1])


def _combine(ps, pr, pi, inv, uinv, iinv):
  return pl.pallas_call(
      _combine_body,
      grid=(NU // _RB,),
      in_specs=[
          pl.BlockSpec((NC, _RB, D), lambda i: (0, i, 0)),
          pl.BlockSpec((NC, _RB, D), lambda i: (0, i, 0)),
          pl.BlockSpec((NC, _RB, D), lambda i: (0, i, 0)),
          pl.BlockSpec((_RB, 1), lambda i: (i, 0)),
          pl.BlockSpec((_RB, 1), lambda i: (i, 0)),
          pl.BlockSpec((_RB, 1), lambda i: (i, 0)),
      ],
      out_specs=[
          pl.BlockSpec((_RB, D), lambda i: (i, 0)),
          pl.BlockSpec((_RB, D), lambda i: (i, 0)),
      ],
      out_shape=[
          jax.ShapeDtypeStruct((NU, D), jnp.float32),
          jax.ShapeDtypeStruct((NI, D), jnp.float32),
      ],
  )(ps, pr, pi, inv, uinv, iinv)


def _final_body(u0, u1, u2, i0, i1, i2, uf, if_):
  c = 1.0 / (N_LAYERS + 1)
  uf[...] = (u0[...] + u1[...] + u2[...]) * c
  if_[...] = (i0[...] + i1[...] + i2[...]) * c


def _final(u0, u1, u2, i0, i1, i2):
  bs = pl.BlockSpec((_RB, D), lambda i: (i, 0))
  return pl.pallas_call(
      _final_body,
      grid=(NU // _RB,),
      in_specs=[bs] * 6,
      out_specs=[bs, bs],
      out_shape=[
          jax.ShapeDtypeStruct((NU, D), jnp.float32),
          jax.ShapeDtypeStruct((NI, D), jnp.float32),
      ],
  )(u0, u1, u2, i0, i1, i2)


def kernel(user_emb, item_emb, W1, b1, W2, b2, social_edge_index, ui_edge_index):
  src = social_edge_index[0].astype(jnp.int32)
  dst = social_edge_index[1].astype(jnp.int32)
  u_idx = ui_edge_index[0].astype(jnp.int32)
  i_idx = ui_edge_index[1].astype(jnp.int32)

  wa = W1[:D]
  wb = W1[D:]
  w2v = W2.reshape(-1)
  consts = jnp.full((16,), 1.0, jnp.float32) * b2[0]
  zeros1d = jnp.zeros((NU,), jnp.float32)
  zrows = jnp.zeros((CHUNK, D), jnp.float32)

  a_tab, b_tab = _ab_matmul(user_emb, wa, wb, b1)

  (g,) = _sc_gate(a_tab, b_tab, w2v, consts, src, dst)

  all_u, all_i = user_emb, item_emb
  layer_u, layer_i = [], []
  inv = uinv = iinv = None
  for layer in range(N_LAYERS):
    if layer == 0:
      ps, rs_p = _sc_segsum_scaled_rs(all_u, dst, src, g, zrows, zeros1d)
      pr, ud_p = _sc_segsum_plain_deg(all_i, i_idx, u_idx, g, zrows, zeros1d)
      pi, id_p = _sc_segsum_plain_deg(all_u, u_idx, i_idx, g, zrows, zeros1d)
      inv, uinv, iinv = _prep(rs_p, ud_p, id_p)
      inv = inv.reshape(NU, 1)
      uinv = uinv.reshape(NU, 1)
      iinv = iinv.reshape(NI, 1)
    else:
      (ps,) = _sc_segsum_scaled(all_u, dst, src, g, zrows)
      (pr,) = _sc_segsum_plain(all_i, i_idx, u_idx, g, zrows)
      (pi,) = _sc_segsum_plain(all_u, u_idx, i_idx, g, zrows)
    all_u, all_i = _combine(ps, pr, pi, inv, uinv, iinv)
    layer_u.append(all_u)
    layer_i.append(all_i)

  uf, if_ = _final(user_emb, layer_u[0], layer_u[1],
                   item_emb, layer_i[0], layer_i[1])
  return jnp.concatenate([uf, if_], axis=0)


# speedup vs baseline: 1.0206x; 1.0206x over previous
import functools

import jax
import jax.numpy as jnp
from jax import lax
from jax.experimental import pallas as pl
from jax.experimental.pallas import tpu as pltpu
from jax.experimental.pallas import tpu_sc as plsc

NU = 10000
NI = 10000
D = 128
NE = 320000
NC = 2
NS = 16
NW = NC * NS
CHUNK = 128
NCHUNKS = NE // CHUNK
ROWS_PER_TILE = NU // NS
EDGE_BIAS = 0.5
N_LAYERS = 2

_mesh = plsc.VectorSubcoreMesh(
    core_axis_name="c", subcore_axis_name="s", num_cores=NC, num_subcores=NS)


def _wid_and_chunks():
  sid = lax.axis_index("s")
  cid = lax.axis_index("c")
  wid = sid * NC + cid
  nfull = NCHUNKS // NW
  rem = NCHUNKS - nfull * NW
  nchunks = nfull + jnp.where(wid < rem, 1, 0)
  return sid, cid, wid, nchunks


_BCH = 26
_NBLK = 3
_BE = _BCH * CHUNK


@functools.partial(
    pl.kernel,
    out_type=[
        jax.ShapeDtypeStruct((NE,), jnp.float32),
    ],
    mesh=_mesh,
    compiler_params=pltpu.CompilerParams(needs_layout_passes=False),
    scratch_types=[
        pltpu.VMEM((_BE,), jnp.int32),
        pltpu.VMEM((_BE,), jnp.int32),
        pltpu.VMEM((_BE,), jnp.float32),
        pltpu.VMEM((CHUNK, D), jnp.float32),
        pltpu.VMEM((CHUNK, D), jnp.float32),
        pltpu.VMEM((CHUNK, D), jnp.float32),
        pltpu.VMEM((CHUNK, D), jnp.float32),
        pltpu.VMEM((D,), jnp.float32),
        pltpu.VMEM((256,), jnp.float32),
        pltpu.VMEM((16,), jnp.float32),
        pltpu.SemaphoreType.DMA,
        pltpu.SemaphoreType.DMA,
    ],
)
def _sc_gate(a_hbm, b_hbm, w2_hbm, consts_hbm, src_hbm, dst_hbm,
             g_hbm,
             srcb, dstb, gblk, bufA0, bufB0, bufA1, bufB1,
             w2_buf, dot_buf, cbuf, sem0, sem1):
  sid, cid, wid, _ = _wid_and_chunks()
  cstart = wid * 78 + jnp.minimum(wid, 4)
  has_tail = wid < 4

  pltpu.sync_copy(w2_hbm, w2_buf)
  pltpu.sync_copy(consts_hbm, cbuf)

  b2vec = cbuf[...]
  iota16 = lax.iota(jnp.int32, 16)
  w2vecs = [w2_buf[pl.ds(kb * 16, 16)] for kb in range(8)]

  def issue(jrel, bA, bB, sem):
    off = pl.ds(jrel * CHUNK, CHUNK)
    pltpu.async_copy(a_hbm.at[srcb.at[off]], bA, sem)
    pltpu.async_copy(b_hbm.at[dstb.at[off]], bB, sem)

  def wait(bA, bB, sem):
    pltpu.make_async_copy(a_hbm.at[pl.ds(0, CHUNK)], bA, sem).wait()
    pltpu.make_async_copy(b_hbm.at[pl.ds(0, CHUNK)], bB, sem).wait()

  def compute(bA, bB, jrel):
    goff = jrel * CHUNK
    for gr in range(8):

      def e_body(t4, carry2):
        for q in range(4):
          t = t4 * 4 + q
          e = gr * 16 + t
          acc = jnp.zeros((16,), jnp.float32)
          for kb in range(8):
            sl = pl.ds(kb * 16, 16)
            h = jnp.maximum(bA[e, sl] + bB[e, sl], 0.0)
            acc = acc + h * w2vecs[kb]
          dot_buf[pl.ds(t * 16, 16)] = acc
        return carry2

      lax.fori_loop(0, 4, e_body, 0)
      tot = jnp.zeros((16,), jnp.float32)
      for t2 in range(16):
        tot = tot + plsc.load_gather(dot_buf, [iota16 * 16 + t2])
      logit5 = (tot + b2vec) * 5.0
      g = 1.0 / (1.0 + jnp.exp(-logit5)) + EDGE_BIAS
      gblk[pl.ds(goff + gr * 16, 16)] = g

  def block_body(b, carry):
    bc = cstart + b * _BCH
    pltpu.sync_copy(src_hbm.at[pl.ds(bc * CHUNK, _BE)], srcb)
    pltpu.sync_copy(dst_hbm.at[pl.ds(bc * CHUNK, _BE)], dstb)
    issue(0, bufA0, bufB0, sem0)

    def pair_body(p, c2):
      j0 = p * 2
      j1 = j0 + 1
      issue(j1, bufA1, bufB1, sem1)
      wait(bufA0, bufB0, sem0)
      compute(bufA0, bufB0, j0)

      @pl.when(j1 + 1 < _BCH)
      def _nxt():
        issue(j1 + 1, bufA0, bufB0, sem0)

      wait(bufA1, bufB1, sem1)
      compute(bufA1, bufB1, j1)
      return c2

    lax.fori_loop(0, _BCH // 2, pair_body, 0)
    pltpu.sync_copy(gblk, g_hbm.at[pl.ds(bc * CHUNK, _BE)])
    return carry

  lax.fori_loop(0, _NBLK, block_body, 0)

  @pl.when(has_tail)
  def _tail():
    tb = (cstart + _NBLK * _BCH) * CHUNK
    pltpu.sync_copy(src_hbm.at[pl.ds(tb, CHUNK)], srcb.at[pl.ds(0, CHUNK)])
    pltpu.sync_copy(dst_hbm.at[pl.ds(tb, CHUNK)], dstb.at[pl.ds(0, CHUNK)])
    issue(0, bufA0, bufB0, sem0)
    wait(bufA0, bufB0, sem0)
    compute(bufA0, bufB0, 0)
    pltpu.sync_copy(gblk.at[pl.ds(0, CHUNK)], g_hbm.at[pl.ds(tb, CHUNK)])


def _make_segsum(scaled: bool, extra: str | None = None):
  with_extra = extra is not None
  BCH = 26
  NBLK = 3
  BE = BCH * CHUNK

  scratch = [
      pltpu.VMEM((BE,), jnp.int32),
      pltpu.VMEM((CHUNK,), jnp.int32),
      pltpu.VMEM((CHUNK,), jnp.int32),
      pltpu.VMEM((CHUNK, D), jnp.float32),
      pltpu.VMEM((CHUNK, D), jnp.float32),
      pltpu.VMEM((BE,), jnp.float32),
      pltpu.VMEM((CHUNK,), jnp.float32),
      pltpu.VMEM_SHARED((NU, D), jnp.float32),
  ] + ([pltpu.VMEM_SHARED((NU,), jnp.float32)] if with_extra else []) + [
      pltpu.SemaphoreType.DMA,
      pltpu.SemaphoreType.DMA,
  ]

  NUNITS = NU // CHUNK

  def body(*refs):
    (table_hbm, gidx_hbm, seg_hbm, g_all_hbm, zrows_hbm) = refs[:5]
    refs = refs[5:]
    if with_extra:
      zeros1d_hbm, part_out, extra_out = refs[:3]
      refs = refs[3:]
    else:
      part_out = refs[0]
      refs = refs[1:]
    gidx, segb0, segb1, rows0, rows1, gblk, ones_buf, acc = refs[:8]
    refs = refs[8:]
    if with_extra:
      extra_acc = refs[0]
      refs = refs[1:]
    sem0, sem1 = refs
    sid, cid, wid, _ = _wid_and_chunks()
    nunits = NUNITS // NS + jnp.where(sid < NUNITS % NS, 1, 0)
    cstart = wid * 78 + jnp.minimum(wid, 4)
    has_tail = wid < 4

    pltpu.sync_copy(zrows_hbm, rows0)

    def zero_body(k, c2):
      u0 = (sid + k * NS) * CHUNK
      pltpu.sync_copy(rows0, acc.at[pl.ds(u0, CHUNK)])
      return c2

    lax.fori_loop(0, nunits, zero_body, 0)

    @pl.when(sid == 15)
    def _ztail():
      pltpu.sync_copy(rows0.at[pl.ds(0, 16)], acc.at[pl.ds(NUNITS * CHUNK, 16)])

    if with_extra:
      if extra == 'deg':
        for kb in range(8):
          ones_buf[pl.ds(kb * 16, 16)] = jnp.ones((16,), jnp.float32)

      @pl.when(sid == 0)
      def _zextra():
        pltpu.sync_copy(zeros1d_hbm, extra_acc)

    plsc.subcore_barrier()

    def extra_scatter(segb, jrel):
      if extra == 'deg':
        pltpu.sync_copy(ones_buf, extra_acc.at[segb], add=True)
      elif extra == 'rs':
        pltpu.sync_copy(gblk.at[pl.ds(jrel * CHUNK, CHUNK)],
                        extra_acc.at[segb], add=True)

    def issue(bc, jrel, rbuf, sem):
      pltpu.async_copy(table_hbm.at[gidx.at[pl.ds(jrel * CHUNK, CHUNK)]],
                       rbuf, sem)

    def wait(rbuf, sem):
      pltpu.make_async_copy(table_hbm.at[pl.ds(0, CHUNK)], rbuf, sem).wait()

    def scale(rbuf, jrel):
      if not scaled:
        return
      goff = jrel * CHUNK

      def scale_body(p, c2):
        for q in range(4):
          e = p * 4 + q
          ge = plsc.load_gather(gblk, [jnp.full((16,), 0, jnp.int32) + goff + e])
          for kb in range(8):
            sl = pl.ds(kb * 16, 16)
            rbuf[e, sl] = rbuf[e, sl] * ge
        return c2

      lax.fori_loop(0, CHUNK // 4, scale_body, 0)

    def block_body(b, carry):
      bc = cstart + b * BCH
      pltpu.sync_copy(gidx_hbm.at[pl.ds(bc * CHUNK, BE)], gidx)
      if scaled:
        pltpu.sync_copy(g_all_hbm.at[pl.ds(bc * CHUNK, BE)], gblk)
      issue(bc, 0, rows0, sem0)

      def pair_body(p, c2):
        j0 = p * 2
        j1 = j0 + 1
        issue(bc, j1, rows1, sem1)
        pltpu.sync_copy(seg_hbm.at[pl.ds((bc + j0) * CHUNK, CHUNK)], segb0)
        wait(rows0, sem0)
        scale(rows0, j0)
        pltpu.sync_copy(rows0, acc.at[segb0], add=True)
        if with_extra:
          extra_scatter(segb0, j0)

        @pl.when(j1 + 1 < BCH)
        def _nxt():
          issue(bc, j1 + 1, rows0, sem0)

        pltpu.sync_copy(seg_hbm.at[pl.ds((bc + j1) * CHUNK, CHUNK)], segb1)
        wait(rows1, sem1)
        scale(rows1, j1)
        pltpu.sync_copy(rows1, acc.at[segb1], add=True)
        if with_extra:
          extra_scatter(segb1, j1)
        return c2

      lax.fori_loop(0, BCH // 2, pair_body, 0)
      return carry

    lax.fori_loop(0, NBLK, block_body, 0)

    @pl.when(has_tail)
    def _tail():
      tb = (cstart + NBLK * BCH) * CHUNK
      pltpu.sync_copy(gidx_hbm.at[pl.ds(tb, CHUNK)], gidx.at[pl.ds(0, CHUNK)])
      if scaled:
        pltpu.sync_copy(g_all_hbm.at[pl.ds(tb, CHUNK)], gblk.at[pl.ds(0, CHUNK)])
      issue(0, 0, rows0, sem0)
      pltpu.sync_copy(seg_hbm.at[pl.ds(tb, CHUNK)], segb0)
      wait(rows0, sem0)
      scale(rows0, 0)
      pltpu.sync_copy(rows0, acc.at[segb0], add=True)
      if with_extra:
        extra_scatter(segb0, 0)

    plsc.subcore_barrier()

    def drain_body(k, c2):
      u0 = (sid + k * NS) * CHUNK
      pltpu.sync_copy(acc.at[pl.ds(u0, CHUNK)], rows0)
      pltpu.sync_copy(rows0, part_out.at[cid, pl.ds(u0, CHUNK)])
      return c2

    lax.fori_loop(0, nunits, drain_body, 0)

    @pl.when(sid == 15)
    def _dtail():
      pltpu.sync_copy(acc.at[pl.ds(NUNITS * CHUNK, 16)], rows0.at[pl.ds(0, 16)])
      pltpu.sync_copy(rows0.at[pl.ds(0, 16)],
                      part_out.at[cid, pl.ds(NUNITS * CHUNK, 16)])

    if with_extra:
      @pl.when(sid == 0)
      def _dextra():
        pltpu.sync_copy(extra_acc, extra_out.at[cid])

  out_type = [jax.ShapeDtypeStruct((NC, NU, D), jnp.float32)]
  if with_extra:
    out_type.append(jax.ShapeDtypeStruct((NC, NU), jnp.float32))
  return functools.partial(
      pl.kernel,
      out_type=out_type,
      mesh=_mesh,
      compiler_params=pltpu.CompilerParams(needs_layout_passes=False),
      scratch_types=scratch,
  )(body)


_sc_segsum_scaled = _make_segsum(True)
_sc_segsum_scaled_rs = _make_segsum(True, extra='rs')
_sc_segsum_plain = _make_segsum(False)
_sc_segsum_plain_deg = _make_segsum(False, extra='deg')


_RB = 1000


def _ab_body(x_ref, wa_ref, wb_ref, b1_ref, a_ref, b_ref):
  x = x_ref[...]
  a_ref[...] = jnp.dot(x, wa_ref[...],
                       preferred_element_type=jnp.float32) + b1_ref[...][None, :]
  b_ref[...] = jnp.dot(x, wb_ref[...], preferred_element_type=jnp.float32)


def _ab_matmul(x, wa, wb, b1):
  return pl.pallas_call(
      _ab_body,
      grid=(NU // _RB,),
      in_specs=[
          pl.BlockSpec((_RB, D), lambda i: (i, 0)),
          pl.BlockSpec((D, D), lambda i: (0, 0)),
          pl.BlockSpec((D, D), lambda i: (0, 0)),
          pl.BlockSpec((D,), lambda i: (0,)),
      ],
      out_specs=[
          pl.BlockSpec((_RB, D), lambda i: (i, 0)),
          pl.BlockSpec((_RB, D), lambda i: (i, 0)),
      ],
      out_shape=[
          jax.ShapeDtypeStruct((NU, D), jnp.float32),
          jax.ShapeDtypeStruct((NU, D), jnp.float32),
      ],
  )(x, wa, wb, b1)


def _prep_body(rs_ref, ud_ref, id_ref, inv_ref, uinv_ref, iinv_ref):
  def inv_of(ref):
    t = ref[0, :] + ref[1, :]
    return jnp.where(t > 0, 1.0 / t, 0.0)

  inv_ref[...] = inv_of(rs_ref)
  uinv_ref[...] = inv_of(ud_ref)
  iinv_ref[...] = inv_of(id_ref)


def _prep(rs_p, ud_p, id_p):
  return pl.pallas_call(
      _prep_body,
      out_shape=[
          jax.ShapeDtypeStruct((NU,), jnp.float32),
          jax.ShapeDtypeStruct((NU,), jnp.float32),
          jax.ShapeDtypeStruct((NI,), jnp.float32),
      ],
  )(rs_p, ud_p, id_p)


def _combine_body(ps_ref, pr_ref, pi_ref, inv_ref, uinv_ref, iinv_ref,
                  u_ref, i_ref):
  u_ref[...] = (inv_ref[...] * (ps_ref[0] + ps_ref[1])
                + uinv_ref[...] * (pr_ref[0] + pr_ref[1]))
  i_ref[...] = iinv_ref[...] * (pi_ref[0] + pi_ref[1])


def _combine(ps, pr, pi, inv, uinv, iinv):
  return pl.pallas_call(
      _combine_body,
      grid=(NU // _RB,),
      in_specs=[
          pl.BlockSpec((NC, _RB, D), lambda i: (0, i, 0)),
          pl.BlockSpec((NC, _RB, D), lambda i: (0, i, 0)),
          pl.BlockSpec((NC, _RB, D), lambda i: (0, i, 0)),
          pl.BlockSpec((_RB, 1), lambda i: (i, 0)),
          pl.BlockSpec((_RB, 1), lambda i: (i, 0)),
          pl.BlockSpec((_RB, 1), lambda i: (i, 0)),
      ],
      out_specs=[
          pl.BlockSpec((_RB, D), lambda i: (i, 0)),
          pl.BlockSpec((_RB, D), lambda i: (i, 0)),
      ],
      out_shape=[
          jax.ShapeDtypeStruct((NU, D), jnp.float32),
          jax.ShapeDtypeStruct((NI, D), jnp.float32),
      ],
  )(ps, pr, pi, inv, uinv, iinv)


def _final_body(u0, u1, u2, i0, i1, i2, uf, if_):
  c = 1.0 / (N_LAYERS + 1)
  uf[...] = (u0[...] + u1[...] + u2[...]) * c
  if_[...] = (i0[...] + i1[...] + i2[...]) * c


def _final(u0, u1, u2, i0, i1, i2):
  bs = pl.BlockSpec((_RB, D), lambda i: (i, 0))
  return pl.pallas_call(
      _final_body,
      grid=(NU // _RB,),
      in_specs=[bs] * 6,
      out_specs=[bs, bs],
      out_shape=[
          jax.ShapeDtypeStruct((NU, D), jnp.float32),
          jax.ShapeDtypeStruct((NI, D), jnp.float32),
      ],
  )(u0, u1, u2, i0, i1, i2)


def kernel(user_emb, item_emb, W1, b1, W2, b2, social_edge_index, ui_edge_index):
  src = social_edge_index[0].astype(jnp.int32)
  dst = social_edge_index[1].astype(jnp.int32)
  u_idx = ui_edge_index[0].astype(jnp.int32)
  i_idx = ui_edge_index[1].astype(jnp.int32)

  wa = W1[:D]
  wb = W1[D:]
  w2v = W2.reshape(-1)
  consts = jnp.full((16,), 1.0, jnp.float32) * b2[0]
  zeros1d = jnp.zeros((NU,), jnp.float32)
  zrows = jnp.zeros((CHUNK, D), jnp.float32)

  a_tab, b_tab = _ab_matmul(user_emb, wa, wb, b1)

  (g,) = _sc_gate(a_tab, b_tab, w2v, consts, src, dst)

  all_u, all_i = user_emb, item_emb
  layer_u, layer_i = [], []
  inv = uinv = iinv = None
  for layer in range(N_LAYERS):
    if layer == 0:
      ps, rs_p = _sc_segsum_scaled_rs(all_u, dst, src, g, zrows, zeros1d)
      pr, ud_p = _sc_segsum_plain_deg(all_i, i_idx, u_idx, g, zrows, zeros1d)
      pi, id_p = _sc_segsum_plain_deg(all_u, u_idx, i_idx, g, zrows, zeros1d)
      inv, uinv, iinv = _prep(rs_p, ud_p, id_p)
      inv = inv.reshape(NU, 1)
      uinv = uinv.reshape(NU, 1)
      iinv = iinv.reshape(NI, 1)
    else:
      (ps,) = _sc_segsum_scaled(all_u, dst, src, g, zrows)
      (pr,) = _sc_segsum_plain(all_i, i_idx, u_idx, g, zrows)
      (pi,) = _sc_segsum_plain(all_u, u_idx, i_idx, g, zrows)
    all_u, all_i = _combine(ps, pr, pi, inv, uinv, iinv)
    layer_u.append(all_u)
    layer_i.append(all_i)

  uf, if_ = _final(user_emb, layer_u[0], layer_u[1],
                   item_emb, layer_i[0], layer_i[1])
  return jnp.concatenate([uf, if_], axis=0)


# speedup vs baseline: 1.0253x; 1.0046x over previous
import functools

import jax
import jax.numpy as jnp
from jax import lax
from jax.experimental import pallas as pl
from jax.experimental.pallas import tpu as pltpu
from jax.experimental.pallas import tpu_sc as plsc

NU = 10000
NI = 10000
D = 128
NE = 320000
NC = 2
NS = 16
NW = NC * NS
CHUNK = 128
NCHUNKS = NE // CHUNK
ROWS_PER_TILE = NU // NS
EDGE_BIAS = 0.5
N_LAYERS = 2

_mesh = plsc.VectorSubcoreMesh(
    core_axis_name="c", subcore_axis_name="s", num_cores=NC, num_subcores=NS)


def _wid_and_chunks():
  sid = lax.axis_index("s")
  cid = lax.axis_index("c")
  wid = sid * NC + cid
  nfull = NCHUNKS // NW
  rem = NCHUNKS - nfull * NW
  nchunks = nfull + jnp.where(wid < rem, 1, 0)
  return sid, cid, wid, nchunks


_BCH = 26
_NBLK = 3
_BE = _BCH * CHUNK


@functools.partial(
    pl.kernel,
    out_type=[
        jax.ShapeDtypeStruct((NE,), jnp.float32),
    ],
    mesh=_mesh,
    compiler_params=pltpu.CompilerParams(needs_layout_passes=False),
    scratch_types=[
        pltpu.VMEM((_BE,), jnp.int32),
        pltpu.VMEM((_BE,), jnp.int32),
        pltpu.VMEM((_BE,), jnp.float32),
        pltpu.VMEM((CHUNK, D), jnp.float32),
        pltpu.VMEM((CHUNK, D), jnp.float32),
        pltpu.VMEM((CHUNK, D), jnp.float32),
        pltpu.VMEM((CHUNK, D), jnp.float32),
        pltpu.VMEM((D,), jnp.float32),
        pltpu.VMEM((256,), jnp.float32),
        pltpu.VMEM((16,), jnp.float32),
        pltpu.SemaphoreType.DMA,
        pltpu.SemaphoreType.DMA,
    ],
)
def _sc_gate(a_hbm, b_hbm, w2_hbm, consts_hbm, src_hbm, dst_hbm,
             g_hbm,
             srcb, dstb, gblk, bufA0, bufB0, bufA1, bufB1,
             w2_buf, dot_buf, cbuf, sem0, sem1):
  sid, cid, wid, _ = _wid_and_chunks()
  cstart = wid * 78 + jnp.minimum(wid, 4)
  has_tail = wid < 4

  pltpu.sync_copy(w2_hbm, w2_buf)
  pltpu.sync_copy(consts_hbm, cbuf)

  b2vec = cbuf[...]
  iota16 = lax.iota(jnp.int32, 16)
  w2vecs = [w2_buf[pl.ds(kb * 16, 16)] for kb in range(8)]

  def issue(jrel, bA, bB, sem):
    off = pl.ds(jrel * CHUNK, CHUNK)
    pltpu.async_copy(a_hbm.at[srcb.at[off]], bA, sem)
    pltpu.async_copy(b_hbm.at[dstb.at[off]], bB, sem)

  def wait(bA, bB, sem):
    pltpu.make_async_copy(a_hbm.at[pl.ds(0, CHUNK)], bA, sem).wait()
    pltpu.make_async_copy(b_hbm.at[pl.ds(0, CHUNK)], bB, sem).wait()

  def compute(bA, bB, jrel):
    goff = jrel * CHUNK
    for gr in range(8):

      def e_body(t4, carry2):
        for q in range(4):
          t = t4 * 4 + q
          e = gr * 16 + t
          acc = jnp.zeros((16,), jnp.float32)
          for kb in range(8):
            sl = pl.ds(kb * 16, 16)
            h = jnp.maximum(bA[e, sl] + bB[e, sl], 0.0)
            acc = acc + h * w2vecs[kb]
          dot_buf[pl.ds(t * 16, 16)] = acc
        return carry2

      lax.fori_loop(0, 4, e_body, 0)
      tot = jnp.zeros((16,), jnp.float32)
      for t2 in range(16):
        tot = tot + plsc.load_gather(dot_buf, [iota16 * 16 + t2])
      logit5 = (tot + b2vec) * 5.0
      g = 1.0 / (1.0 + jnp.exp(-logit5)) + EDGE_BIAS
      gblk[pl.ds(goff + gr * 16, 16)] = g

  def block_body(b, carry):
    bc = cstart + b * _BCH
    pltpu.sync_copy(src_hbm.at[pl.ds(bc * CHUNK, _BE)], srcb)
    pltpu.sync_copy(dst_hbm.at[pl.ds(bc * CHUNK, _BE)], dstb)
    issue(0, bufA0, bufB0, sem0)

    def pair_body(p, c2):
      j0 = p * 2
      j1 = j0 + 1
      issue(j1, bufA1, bufB1, sem1)
      wait(bufA0, bufB0, sem0)
      compute(bufA0, bufB0, j0)

      @pl.when(j1 + 1 < _BCH)
      def _nxt():
        issue(j1 + 1, bufA0, bufB0, sem0)

      wait(bufA1, bufB1, sem1)
      compute(bufA1, bufB1, j1)
      return c2

    lax.fori_loop(0, _BCH // 2, pair_body, 0)
    pltpu.sync_copy(gblk, g_hbm.at[pl.ds(bc * CHUNK, _BE)])
    return carry

  lax.fori_loop(0, _NBLK, block_body, 0)

  @pl.when(has_tail)
  def _tail():
    tb = (cstart + _NBLK * _BCH) * CHUNK
    pltpu.sync_copy(src_hbm.at[pl.ds(tb, CHUNK)], srcb.at[pl.ds(0, CHUNK)])
    pltpu.sync_copy(dst_hbm.at[pl.ds(tb, CHUNK)], dstb.at[pl.ds(0, CHUNK)])
    issue(0, bufA0, bufB0, sem0)
    wait(bufA0, bufB0, sem0)
    compute(bufA0, bufB0, 0)
    pltpu.sync_copy(gblk.at[pl.ds(0, CHUNK)], g_hbm.at[pl.ds(tb, CHUNK)])


def _make_segsum(scaled: bool, extra: str | None = None):
  with_extra = extra is not None
  BCH = 26
  NBLK = 3
  BE = BCH * CHUNK

  scratch = [
      pltpu.VMEM((BE,), jnp.int32),
      pltpu.VMEM((CHUNK,), jnp.int32),
      pltpu.VMEM((CHUNK,), jnp.int32),
      pltpu.VMEM((CHUNK, D), jnp.float32),
      pltpu.VMEM((CHUNK, D), jnp.float32),
      pltpu.VMEM((BE,), jnp.float32),
      pltpu.VMEM((CHUNK,), jnp.float32),
      pltpu.VMEM_SHARED((NU, D), jnp.float32),
  ] + ([pltpu.VMEM_SHARED((NU,), jnp.float32)] if with_extra else []) + [
      pltpu.SemaphoreType.DMA,
      pltpu.SemaphoreType.DMA,
  ]

  NUNITS = NU // CHUNK

  def body(*refs):
    (table_hbm, gidx_hbm, seg_hbm, g_all_hbm, zrows_hbm) = refs[:5]
    refs = refs[5:]
    if with_extra:
      zeros1d_hbm, part_out, extra_out = refs[:3]
      refs = refs[3:]
    else:
      part_out = refs[0]
      refs = refs[1:]
    gidx, segb0, segb1, rows0, rows1, gblk, ones_buf, acc = refs[:8]
    refs = refs[8:]
    if with_extra:
      extra_acc = refs[0]
      refs = refs[1:]
    sem0, sem1 = refs
    sid, cid, wid, _ = _wid_and_chunks()
    nunits = NUNITS // NS + jnp.where(sid < NUNITS % NS, 1, 0)
    cstart = wid * 78 + jnp.minimum(wid, 4)
    has_tail = wid < 4

    pltpu.sync_copy(zrows_hbm, rows0)

    def zero_body(k, c2):
      u0 = (sid + k * NS) * CHUNK
      pltpu.sync_copy(rows0, acc.at[pl.ds(u0, CHUNK)])
      return c2

    lax.fori_loop(0, nunits, zero_body, 0)

    @pl.when(sid == 15)
    def _ztail():
      pltpu.sync_copy(rows0.at[pl.ds(0, 16)], acc.at[pl.ds(NUNITS * CHUNK, 16)])

    if with_extra:
      if extra == 'deg':
        for kb in range(8):
          ones_buf[pl.ds(kb * 16, 16)] = jnp.ones((16,), jnp.float32)

      @pl.when(sid == 0)
      def _zextra():
        pltpu.sync_copy(zeros1d_hbm, extra_acc)

    plsc.subcore_barrier()

    def extra_scatter(segb, jrel):
      if extra == 'deg':
        pltpu.sync_copy(ones_buf, extra_acc.at[segb], add=True)
      elif extra == 'rs':
        pltpu.sync_copy(gblk.at[pl.ds(jrel * CHUNK, CHUNK)],
                        extra_acc.at[segb], add=True)

    def issue(bc, jrel, rbuf, sem):
      pltpu.async_copy(table_hbm.at[gidx.at[pl.ds(jrel * CHUNK, CHUNK)]],
                       rbuf, sem)

    def wait(rbuf, sem):
      pltpu.make_async_copy(table_hbm.at[pl.ds(0, CHUNK)], rbuf, sem).wait()

    def scale(rbuf, jrel):
      if not scaled:
        return
      goff = jrel * CHUNK

      def scale_body(p, c2):
        for q in range(4):
          e = p * 4 + q
          ge = plsc.load_gather(gblk, [jnp.full((16,), 0, jnp.int32) + goff + e])
          for kb in range(8):
            sl = pl.ds(kb * 16, 16)
            rbuf[e, sl] = rbuf[e, sl] * ge
        return c2

      lax.fori_loop(0, CHUNK // 4, scale_body, 0)

    def block_body(b, carry):
      bc = cstart + b * BCH
      pltpu.sync_copy(gidx_hbm.at[pl.ds(bc * CHUNK, BE)], gidx)
      if scaled:
        pltpu.sync_copy(g_all_hbm.at[pl.ds(bc * CHUNK, BE)], gblk)
      issue(bc, 0, rows0, sem0)

      def pair_body(p, c2):
        j0 = p * 2
        j1 = j0 + 1
        issue(bc, j1, rows1, sem1)
        pltpu.sync_copy(seg_hbm.at[pl.ds((bc + j0) * CHUNK, CHUNK)], segb0)
        wait(rows0, sem0)
        scale(rows0, j0)
        pltpu.sync_copy(rows0, acc.at[segb0], add=True)
        if with_extra:
          extra_scatter(segb0, j0)

        @pl.when(j1 + 1 < BCH)
        def _nxt():
          issue(bc, j1 + 1, rows0, sem0)

        pltpu.sync_copy(seg_hbm.at[pl.ds((bc + j1) * CHUNK, CHUNK)], segb1)
        wait(rows1, sem1)
        scale(rows1, j1)
        pltpu.sync_copy(rows1, acc.at[segb1], add=True)
        if with_extra:
          extra_scatter(segb1, j1)
        return c2

      lax.fori_loop(0, BCH // 2, pair_body, 0)
      return carry

    lax.fori_loop(0, NBLK, block_body, 0)

    @pl.when(has_tail)
    def _tail():
      tb = (cstart + NBLK * BCH) * CHUNK
      pltpu.sync_copy(gidx_hbm.at[pl.ds(tb, CHUNK)], gidx.at[pl.ds(0, CHUNK)])
      if scaled:
        pltpu.sync_copy(g_all_hbm.at[pl.ds(tb, CHUNK)], gblk.at[pl.ds(0, CHUNK)])
      issue(0, 0, rows0, sem0)
      pltpu.sync_copy(seg_hbm.at[pl.ds(tb, CHUNK)], segb0)
      wait(rows0, sem0)
      scale(rows0, 0)
      pltpu.sync_copy(rows0, acc.at[segb0], add=True)
      if with_extra:
        extra_scatter(segb0, 0)

    plsc.subcore_barrier()

    def drain_body(k, c2):
      u0 = (sid + k * NS) * CHUNK
      pltpu.sync_copy(acc.at[pl.ds(u0, CHUNK)], rows0)
      pltpu.sync_copy(rows0, part_out.at[cid, pl.ds(u0, CHUNK)])
      return c2

    lax.fori_loop(0, nunits, drain_body, 0)

    @pl.when(sid == 15)
    def _dtail():
      pltpu.sync_copy(acc.at[pl.ds(NUNITS * CHUNK, 16)], rows0.at[pl.ds(0, 16)])
      pltpu.sync_copy(rows0.at[pl.ds(0, 16)],
                      part_out.at[cid, pl.ds(NUNITS * CHUNK, 16)])

    if with_extra:
      @pl.when(sid == 0)
      def _dextra():
        pltpu.sync_copy(extra_acc, extra_out.at[cid])

  out_type = [jax.ShapeDtypeStruct((NC, NU, D), jnp.float32)]
  if with_extra:
    out_type.append(jax.ShapeDtypeStruct((NC, NU), jnp.float32))
  return functools.partial(
      pl.kernel,
      out_type=out_type,
      mesh=_mesh,
      compiler_params=pltpu.CompilerParams(needs_layout_passes=False),
      scratch_types=scratch,
  )(body)


_sc_segsum_scaled = _make_segsum(True)
_sc_segsum_scaled_rs = _make_segsum(True, extra='rs')
_sc_segsum_plain = _make_segsum(False)
_sc_segsum_plain_deg = _make_segsum(False, extra='deg')


_RB = 1000


def _ab_body(x_ref, wa_ref, wb_ref, b1_ref, a_ref, b_ref):
  x = x_ref[...]
  a_ref[...] = jnp.dot(x, wa_ref[...],
                       preferred_element_type=jnp.float32) + b1_ref[...][None, :]
  b_ref[...] = jnp.dot(x, wb_ref[...], preferred_element_type=jnp.float32)


def _ab_matmul(x, wa, wb, b1):
  return pl.pallas_call(
      _ab_body,
      grid=(NU // _RB,),
      in_specs=[
          pl.BlockSpec((_RB, D), lambda i: (i, 0)),
          pl.BlockSpec((D, D), lambda i: (0, 0)),
          pl.BlockSpec((D, D), lambda i: (0, 0)),
          pl.BlockSpec((D,), lambda i: (0,)),
      ],
      out_specs=[
          pl.BlockSpec((_RB, D), lambda i: (i, 0)),
          pl.BlockSpec((_RB, D), lambda i: (i, 0)),
      ],
      out_shape=[
          jax.ShapeDtypeStruct((NU, D), jnp.float32),
          jax.ShapeDtypeStruct((NU, D), jnp.float32),
      ],
  )(x, wa, wb, b1)


def _inv_of(ref):
  t = ref[0, :, 0] + ref[1, :, 0]
  return jnp.where(t > 0, 1.0 / t, 0.0)


def _combine1_body(ps_ref, pr_ref, pi_ref, rs_ref, ud_ref, id_ref,
                   u_ref, i_ref, inv_ref, uinv_ref, iinv_ref):
  inv = _inv_of(rs_ref)[:, None]
  uinv = _inv_of(ud_ref)[:, None]
  iinv = _inv_of(id_ref)[:, None]
  inv_ref[...] = inv
  uinv_ref[...] = uinv
  iinv_ref[...] = iinv
  u_ref[...] = inv * (ps_ref[0] + ps_ref[1]) + uinv * (pr_ref[0] + pr_ref[1])
  i_ref[...] = iinv * (pi_ref[0] + pi_ref[1])


def _combine1(ps, pr, pi, rs_p, ud_p, id_p):
  bsp = pl.BlockSpec((NC, _RB, D), lambda i: (0, i, 0))
  bs1 = pl.BlockSpec((NC, _RB, 1), lambda i: (0, i, 0))
  bsv = pl.BlockSpec((_RB, 1), lambda i: (i, 0))
  bso = pl.BlockSpec((_RB, D), lambda i: (i, 0))
  return pl.pallas_call(
      _combine1_body,
      grid=(NU // _RB,),
      in_specs=[bsp, bsp, bsp, bs1, bs1, bs1],
      out_specs=[bso, bso, bsv, bsv, bsv],
      out_shape=[
          jax.ShapeDtypeStruct((NU, D), jnp.float32),
          jax.ShapeDtypeStruct((NI, D), jnp.float32),
          jax.ShapeDtypeStruct((NU, 1), jnp.float32),
          jax.ShapeDtypeStruct((NU, 1), jnp.float32),
          jax.ShapeDtypeStruct((NI, 1), jnp.float32),
      ],
  )(ps, pr, pi, rs_p.reshape(NC, NU, 1), ud_p.reshape(NC, NU, 1),
    id_p.reshape(NC, NI, 1))


def _combine2_body(ps_ref, pr_ref, pi_ref, inv_ref, uinv_ref, iinv_ref,
                   u0_ref, u1_ref, i0_ref, i1_ref, uf_ref, if_ref):
  c = 1.0 / (N_LAYERS + 1)
  u2 = (inv_ref[...] * (ps_ref[0] + ps_ref[1])
        + uinv_ref[...] * (pr_ref[0] + pr_ref[1]))
  i2 = iinv_ref[...] * (pi_ref[0] + pi_ref[1])
  uf_ref[...] = (u0_ref[...] + u1_ref[...] + u2) * c
  if_ref[...] = (i0_ref[...] + i1_ref[...] + i2) * c


def _combine2(ps, pr, pi, inv, uinv, iinv, u0, u1, i0, i1):
  bsp = pl.BlockSpec((NC, _RB, D), lambda i: (0, i, 0))
  bsv = pl.BlockSpec((_RB, 1), lambda i: (i, 0))
  bso = pl.BlockSpec((_RB, D), lambda i: (i, 0))
  return pl.pallas_call(
      _combine2_body,
      grid=(NU // _RB,),
      in_specs=[bsp, bsp, bsp, bsv, bsv, bsv, bso, bso, bso, bso],
      out_specs=[bso, bso],
      out_shape=[
          jax.ShapeDtypeStruct((NU, D), jnp.float32),
          jax.ShapeDtypeStruct((NI, D), jnp.float32),
      ],
  )(ps, pr, pi, inv, uinv, iinv, u0, u1, i0, i1)


def kernel(user_emb, item_emb, W1, b1, W2, b2, social_edge_index, ui_edge_index):
  src = social_edge_index[0].astype(jnp.int32)
  dst = social_edge_index[1].astype(jnp.int32)
  u_idx = ui_edge_index[0].astype(jnp.int32)
  i_idx = ui_edge_index[1].astype(jnp.int32)

  wa = W1[:D]
  wb = W1[D:]
  w2v = W2.reshape(-1)
  consts = jnp.full((16,), 1.0, jnp.float32) * b2[0]
  zeros1d = jnp.zeros((NU,), jnp.float32)
  zrows = jnp.zeros((CHUNK, D), jnp.float32)

  a_tab, b_tab = _ab_matmul(user_emb, wa, wb, b1)

  (g,) = _sc_gate(a_tab, b_tab, w2v, consts, src, dst)

  ps, rs_p = _sc_segsum_scaled_rs(user_emb, dst, src, g, zrows, zeros1d)
  pr, ud_p = _sc_segsum_plain_deg(item_emb, i_idx, u_idx, g, zrows, zeros1d)
  pi, id_p = _sc_segsum_plain_deg(user_emb, u_idx, i_idx, g, zrows, zeros1d)
  u1, i1, inv, uinv, iinv = _combine1(ps, pr, pi, rs_p, ud_p, id_p)

  (ps2,) = _sc_segsum_scaled(u1, dst, src, g, zrows)
  (pr2,) = _sc_segsum_plain(i1, i_idx, u_idx, g, zrows)
  (pi2,) = _sc_segsum_plain(u1, u_idx, i_idx, g, zrows)
  uf, if_ = _combine2(ps2, pr2, pi2, inv, uinv, iinv,
                      user_emb, u1, item_emb, i1)
  return jnp.concatenate([uf, if_], axis=0)
